# Initial kernel scaffold; baseline (speedup 1.0000x reference)
import jax
import jax.numpy as jnp
from jax.experimental import pallas as pl


def kernel(centroids, assignments):
    raise NotImplementedError("write your pallas kernel here")


# speedup vs baseline: 51.5032x; 51.5032x over previous
import functools

import jax
import jax.numpy as jnp
from jax import lax
from jax.experimental import pallas as pl
from jax.experimental.pallas import tpu as pltpu
from jax.experimental.pallas import tpu_sc as plsc

_N_NODES = 100_000
_DIM = 256
_BLOCK = 4
_M = _N_NODES * _DIM // _BLOCK

_NW = 32
_PER_W = _M // _NW
_CHUNK = 10_000
_NCHUNK = _PER_W // _CHUNK
_GROUPS = _CHUNK // 16


def _epq_body(table_hbm, idx_hbm, out_hbm, table_v, idx_v, out_v):
    info = plsc.get_sparse_core_info()
    nc = info.num_cores
    wid = lax.axis_index("s") * nc + lax.axis_index("c")
    base = wid * _PER_W

    pltpu.sync_copy(table_hbm, table_v)

    iota = lax.iota(jnp.int32, 16)
    sidx = [iota * 4 + c for c in range(4)]

    def chunk_body(i, carry):
        cbase = base + i * _CHUNK
        pltpu.sync_copy(idx_hbm.at[pl.ds(cbase, _CHUNK)], idx_v)

        def group_body(g, gcarry):
            a = idx_v[pl.ds(g * 16, 16)]
            f = a * 4
            obase = g * 64
            for c in range(4):
                v = plsc.load_gather(table_v, [f + c])
                plsc.store_scatter(out_v, [sidx[c] + obase], v)
            return gcarry

        lax.fori_loop(0, _GROUPS, group_body, 0)
        pltpu.sync_copy(out_v, out_hbm.at[pl.ds(cbase * 4, _CHUNK * 4)])
        return carry

    lax.fori_loop(0, _NCHUNK, chunk_body, 0)


_epq = functools.partial(
    pl.kernel,
    out_type=jax.ShapeDtypeStruct((_M * _BLOCK,), jnp.float32),
    mesh=plsc.VectorSubcoreMesh(core_axis_name="c", subcore_axis_name="s"),
    scratch_types=[
        pltpu.VMEM((1024,), jnp.float32),
        pltpu.VMEM((_CHUNK,), jnp.int32),
        pltpu.VMEM((_CHUNK * _BLOCK,), jnp.float32),
    ],
    compiler_params=pltpu.CompilerParams(needs_layout_passes=False),
)(_epq_body)


@jax.jit
def kernel(centroids, assignments):
    table = centroids.reshape(-1)
    flat = _epq(table, assignments)
    return flat.reshape(_N_NODES, _DIM)


# speedup vs baseline: 127.8962x; 2.4833x over previous
import functools

import jax
import jax.numpy as jnp
from jax import lax
from jax.experimental import pallas as pl
from jax.experimental.pallas import tpu as pltpu
from jax.experimental.pallas import tpu_sc as plsc

_N_NODES = 100_000
_DIM = 256
_BLOCK = 4
_M = _N_NODES * _DIM // _BLOCK

_NW = 32
_PER_W = _M // _NW
_CHUNK = 10_000
_NCHUNK = _PER_W // _CHUNK
_GROUPS = _CHUNK // 16
_NBUF = 2


def _epq_body(table_hbm, idx_hbm, out_hbm, table_v, idx_v0, idx_v1,
              out_v0, out_v1, sem_in0, sem_in1, sem_out0, sem_out1):
    idx_v = (idx_v0, idx_v1)
    out_v = (out_v0, out_v1)
    sem_in = (sem_in0, sem_in1)
    sem_out = (sem_out0, sem_out1)
    info = plsc.get_sparse_core_info()
    nc = info.num_cores
    wid = lax.axis_index("s") * nc + lax.axis_index("c")
    base = wid * _PER_W

    pltpu.sync_copy(table_hbm, table_v)

    iota = lax.iota(jnp.int32, 16)
    sidx = [iota * 4 + c for c in range(4)]

    def start_in(ci, b):
        pltpu.async_copy(idx_hbm.at[pl.ds(base + ci * _CHUNK, _CHUNK)],
                         idx_v[b], sem_in[b])

    def start_out(ci, b):
        pltpu.async_copy(out_v[b],
                         out_hbm.at[pl.ds((base + ci * _CHUNK) * 4, _CHUNK * 4)],
                         sem_out[b])

    def wait_in(b):
        pltpu.make_async_copy(idx_hbm.at[pl.ds(base, _CHUNK)],
                              idx_v[b], sem_in[b]).wait()

    def wait_out(b):
        pltpu.make_async_copy(out_v[b],
                              out_hbm.at[pl.ds(base * 4, _CHUNK * 4)],
                              sem_out[b]).wait()

    start_in(0, 0)
    start_in(1, 1)

    @pl.loop(0, _NCHUNK, step=_NBUF)
    def _outer(i):
        for b in range(_NBUF):
            ci = i + b
            wait_in(b)

            @pl.when(ci >= _NBUF)
            def _():
                wait_out(b)

            idx_ref = idx_v[b]
            out_ref = out_v[b]

            @plsc.parallel_loop(0, _GROUPS, unroll=8)
            def _group(g):
                a = idx_ref[pl.ds(g * 16, 16)]
                f = a * 4
                obase = g * 64
                for c in range(4):
                    v = plsc.load_gather(table_v, [f + c])
                    plsc.store_scatter(out_ref, [sidx[c] + obase], v)

            start_out(ci, b)

            @pl.when(ci + _NBUF < _NCHUNK)
            def _():
                start_in(ci + _NBUF, b)

    for b in range(_NBUF):
        wait_out(b)


_epq = functools.partial(
    pl.kernel,
    out_type=jax.ShapeDtypeStruct((_M * _BLOCK,), jnp.float32),
    mesh=plsc.VectorSubcoreMesh(core_axis_name="c", subcore_axis_name="s"),
    scratch_types=[
        pltpu.VMEM((1024,), jnp.float32),
        pltpu.VMEM((_CHUNK,), jnp.int32),
        pltpu.VMEM((_CHUNK,), jnp.int32),
        pltpu.VMEM((_CHUNK * _BLOCK,), jnp.float32),
        pltpu.VMEM((_CHUNK * _BLOCK,), jnp.float32),
        pltpu.SemaphoreType.DMA,
        pltpu.SemaphoreType.DMA,
        pltpu.SemaphoreType.DMA,
        pltpu.SemaphoreType.DMA,
    ],
    compiler_params=pltpu.CompilerParams(needs_layout_passes=False),
)(_epq_body)


@jax.jit
def kernel(centroids, assignments):
    table = centroids.reshape(-1)
    flat = _epq(table, assignments)
    return flat.reshape(_N_NODES, _DIM)


# speedup vs baseline: 269.2044x; 2.1049x over previous
import functools

import jax
import jax.numpy as jnp
from jax import lax
from jax.experimental import pallas as pl
from jax.experimental.pallas import tpu as pltpu
from jax.experimental.pallas import tpu_sc as plsc

_N_NODES = 100_000
_DIM = 256
_BLOCK = 4
_M = _N_NODES * _DIM // _BLOCK

_NW = 32
_CROWS = 128
_CHUNK = _CROWS * _DIM // _BLOCK
_NCHUNKS = _N_NODES // _CROWS
_TAIL_W = _NCHUNKS % _NW
_TAIL_ROWS = _N_NODES - _NCHUNKS * _CROWS
_TAIL_CHUNK = _TAIL_ROWS * _DIM // _BLOCK
_MAX_ORD = 26
_NBUF = 2


def _epq_body(table_hbm, idx_hbm, out_hbm, table_v, idx_v0, idx_v1,
              out_v0, out_v1, sem_in0, sem_in1, sem_out0, sem_out1):
    idx_v = (idx_v0, idx_v1)
    out_v = (out_v0, out_v1)
    sem_in = (sem_in0, sem_in1)
    sem_out = (sem_out0, sem_out1)
    info = plsc.get_sparse_core_info()
    nc = info.num_cores
    wid = lax.axis_index("s") * nc + lax.axis_index("c")

    pltpu.sync_copy(table_hbm, table_v)

    iota = lax.iota(jnp.int32, 16)
    col = [[iota * 4 + (64 * r + c) for c in range(4)] for r in range(4)]

    def start_in(j, b):
        pltpu.async_copy(idx_hbm.at[pl.ds(j * _CHUNK, _CHUNK)],
                         idx_v[b], sem_in[b])

    def start_out(j, b):
        pltpu.async_copy(out_v[b], out_hbm.at[pl.ds(j * _CROWS, _CROWS)],
                         sem_out[b])

    def wait_in(b):
        pltpu.make_async_copy(idx_hbm.at[pl.ds(0, _CHUNK)],
                              idx_v[b], sem_in[b]).wait()

    def wait_out(b):
        pltpu.make_async_copy(out_v[b], out_hbm.at[pl.ds(0, _CROWS)],
                              sem_out[b]).wait()

    def compute(b, quads):
        idx_ref = idx_v[b]
        out_ref = out_v[b]

        @plsc.parallel_loop(0, quads, unroll=4)
        def _quad(q):
            row = jnp.full((16,), q, dtype=jnp.int32)
            for r in range(4):
                a = idx_ref[pl.ds((q * 4 + r) * 16, 16)]
                f = a * 4
                for c in range(4):
                    v = plsc.load_gather(table_v, [f + c])
                    plsc.store_scatter(out_ref, [row, col[r][c]], v)

    @pl.when(wid == _TAIL_W)
    def _():
        pltpu.sync_copy(idx_hbm.at[pl.ds(_NCHUNKS * _CHUNK, _TAIL_CHUNK)],
                        idx_v0.at[pl.ds(0, _TAIL_CHUNK)])
        compute(0, _TAIL_ROWS)
        pltpu.sync_copy(out_v0.at[pl.ds(0, _TAIL_ROWS)],
                        out_hbm.at[pl.ds(_NCHUNKS * _CROWS, _TAIL_ROWS)])

    start_in(wid, 0)
    start_in(wid + _NW, 1)

    @pl.loop(0, _MAX_ORD, step=_NBUF)
    def _outer(k0):
        for b in range(_NBUF):
            k = k0 + b
            j = wid + k * _NW

            @pl.when(j < _NCHUNKS)
            def _():
                wait_in(b)

                @pl.when(k >= _NBUF)
                def _():
                    wait_out(b)

                compute(b, _CROWS)
                start_out(j, b)

                @pl.when(j + _NBUF * _NW < _NCHUNKS)
                def _():
                    start_in(j + _NBUF * _NW, b)

    wait_out(0)
    wait_out(1)


_epq = functools.partial(
    pl.kernel,
    out_type=jax.ShapeDtypeStruct((_N_NODES, _DIM), jnp.float32),
    mesh=plsc.VectorSubcoreMesh(core_axis_name="c", subcore_axis_name="s"),
    scratch_types=[
        pltpu.VMEM((1024,), jnp.float32),
        pltpu.VMEM((_CHUNK,), jnp.int32),
        pltpu.VMEM((_CHUNK,), jnp.int32),
        pltpu.VMEM((_CROWS, _DIM), jnp.float32),
        pltpu.VMEM((_CROWS, _DIM), jnp.float32),
        pltpu.SemaphoreType.DMA,
        pltpu.SemaphoreType.DMA,
        pltpu.SemaphoreType.DMA,
        pltpu.SemaphoreType.DMA,
    ],
    compiler_params=pltpu.CompilerParams(needs_layout_passes=False),
)(_epq_body)


@jax.jit
def kernel(centroids, assignments):
    table = centroids.reshape(-1)
    return _epq(table, assignments)
